# baseline (speedup 1.0000x reference)
import functools
import math

import jax
import jax.numpy as jnp
from jax import lax
from jax.experimental import pallas as pl
from jax.experimental.pallas import tpu as pltpu
from jax.experimental.pallas import tpu_sc as plsc

jax.config.update("jax_enable_x64", True)

NC = 2
NS = 16
L = 16
CH = 512


def _sc_scatter_build(N1, NROWS, CPT, KB):
  RN = N1 // NS
  G = CPT // KB

  mesh = plsc.VectorSubcoreMesh(
      core_axis_name="c", subcore_axis_name="s", num_cores=NC,
      num_subcores=NS)

  @functools.partial(
      pl.kernel,
      out_type=(
          jax.ShapeDtypeStruct((NC, N1), jnp.float32),
          jax.ShapeDtypeStruct((NC, N1), jnp.float32),
      ),
      mesh=mesh,
      compiler_params=pltpu.CompilerParams(
          needs_layout_passes=False, use_tc_tiling_on_sc=False),
      scratch_types=[
          pltpu.VMEM_SHARED((N1, 2), jnp.float32),
          pltpu.VMEM_SHARED((N1, 2), jnp.float32),
          pltpu.VMEM((RN, 2), jnp.float32),
          pltpu.VMEM((RN,), jnp.float32),
          pltpu.VMEM((RN,), jnp.float32),
          pltpu.VMEM((KB, CH), jnp.int32),
          pltpu.VMEM((KB, CH), jnp.int32),
          pltpu.VMEM((CH, 2), jnp.float32),
          pltpu.VMEM((RN,), jnp.float32),
          pltpu.VMEM((RN,), jnp.float32),
      ],
  )
  def sc_kernel(bel_hbm, pay_hbm, src_hbm, dst_hbm, zz_hbm,
                t_out, c_out,
                table_sh, acc_sh, nod_v, bel_v, pay_v,
                srcb, dstb, msg_v, tv_v, cv_v):
    c = lax.axis_index("c")
    s = lax.axis_index("s")
    wid = c * NS + s
    base_n = s * RN

    pltpu.sync_copy(zz_hbm.at[pl.ds(base_n, RN)], nod_v)
    pltpu.sync_copy(nod_v, acc_sh.at[pl.ds(base_n, RN)])
    pltpu.sync_copy(bel_hbm.at[pl.ds(base_n, RN)], bel_v)
    pltpu.sync_copy(pay_hbm.at[pl.ds(base_n, RN)], pay_v)

    iota = lax.iota(jnp.int32, L)
    col0 = jnp.zeros((L,), jnp.int32)
    col1 = jnp.ones((L,), jnp.int32)

    def build(_, off):
      b16 = bel_v[pl.ds(off, L)]
      p16 = pay_v[pl.ds(off, L)]
      m = jnp.maximum(jnp.sign(b16 - 0.5), 0.0)
      t = (2.0 * p16 - 10.0) * m
      rows = off + iota
      plsc.store_scatter(nod_v, [rows, col0], t)
      plsc.store_scatter(nod_v, [rows, col1], m)
      return off + L

    lax.fori_loop(jnp.int32(0), jnp.int32(RN // L), build, jnp.int32(0))
    pltpu.sync_copy(nod_v, table_sh.at[pl.ds(base_n, RN)])
    plsc.subcore_barrier()

    tile_row = wid * CPT

    def group(_, row0):
      row0a = pl.multiple_of(row0, 8)
      pltpu.sync_copy(src_hbm.at[pl.ds(row0a, KB)], srcb)
      pltpu.sync_copy(dst_hbm.at[pl.ds(row0a, KB)], dstb)

      def chunk(_, j):
        pltpu.sync_copy(table_sh.at[srcb.at[j]], msg_v)
        pltpu.sync_copy(msg_v, acc_sh.at[dstb.at[j]], add=True)
        return j + 1

      lax.fori_loop(jnp.int32(0), jnp.int32(KB), chunk, jnp.int32(0))
      return row0 + KB

    lax.fori_loop(jnp.int32(0), jnp.int32(G), group, tile_row)
    plsc.subcore_barrier()

    pltpu.sync_copy(acc_sh.at[pl.ds(base_n, RN)], nod_v)

    def deint(_, off):
      rows = off + iota
      tv_v[pl.ds(off, L)] = plsc.load_gather(nod_v, [rows, col0])
      cv_v[pl.ds(off, L)] = plsc.load_gather(nod_v, [rows, col1])
      return off + L

    lax.fori_loop(jnp.int32(0), jnp.int32(RN // L), deint, jnp.int32(0))
    pltpu.sync_copy(tv_v, t_out.at[c, pl.ds(base_n, RN)])
    pltpu.sync_copy(cv_v, c_out.at[c, pl.ds(base_n, RN)])

  return sc_kernel


def _tc_apply(b_ref, q_ref, t0_ref, t1_ref, c0_ref, c1_ref, o_ref):
  t = t0_ref[...] + t1_ref[...]
  cnt = c0_ref[...] + c1_ref[...]
  b = b_ref[...]
  q = q_ref[...]
  d = t * (jnp.log(1.0 - q) - jnp.log(q))
  den = b + (1.0 - b) * jnp.exp(d)
  post = b / jnp.maximum(den, 1e-35)
  o_ref[...] = jnp.where(cnt > 0.0, post, b)


def kernel(belief, probability, payoff_sample, edge_index):
  N = belief.shape[0]
  E = edge_index.shape[1]

  N1 = ((N + 1023) // 1024) * 1024
  CPT = 8 * math.ceil(E / (NC * NS * CH * 8))
  E1 = CPT * NC * NS * CH
  KB = max(k for k in range(8, 33, 8) if CPT % k == 0)

  f32 = jnp.float32
  bel_p = jnp.concatenate([belief.astype(f32), jnp.zeros((N1 - N,), f32)])
  q_p = jnp.concatenate(
      [probability.astype(f32), jnp.full((N1 - N,), 0.5, f32)])
  pay_p = jnp.concatenate(
      [payoff_sample.astype(f32), jnp.zeros((N1 - N,), f32)])
  pad_e = jnp.full((E1 - E,), N, jnp.int32)
  src2 = jnp.concatenate([edge_index[0].astype(jnp.int32), pad_e])
  dst2 = jnp.concatenate([edge_index[1].astype(jnp.int32), pad_e])
  src2 = src2.reshape(E1 // CH, CH)
  dst2 = dst2.reshape(E1 // CH, CH)
  zz = jnp.zeros((N1, 2), f32)

  sc = _sc_scatter_build(N1, E1 // CH, CPT, KB)
  t_p, c_p = sc(bel_p, pay_p, src2, dst2, zz)

  R = N1 // 128
  GB = max(g for g in range(1, 17)
           if R % g == 0 and (R // g) % 8 == 0)
  blk = (R // GB, 128)
  spec = pl.BlockSpec(blk, lambda i: (i, jnp.int32(0)))
  out = pl.pallas_call(
      _tc_apply,
      grid=(GB,),
      in_specs=[spec] * 6,
      out_specs=spec,
      out_shape=jax.ShapeDtypeStruct((R, 128), f32),
  )(bel_p.reshape(R, 128), q_p.reshape(R, 128),
    t_p[0].reshape(R, 128), t_p[1].reshape(R, 128),
    c_p[0].reshape(R, 128), c_p[1].reshape(R, 128))

  return out.reshape(N1)[:N].astype(jnp.float64)


# speedup vs baseline: 1.2608x; 1.2608x over previous
import functools
import math

import jax
import jax.numpy as jnp
from jax import lax
from jax.experimental import pallas as pl
from jax.experimental.pallas import tpu as pltpu
from jax.experimental.pallas import tpu_sc as plsc

jax.config.update("jax_enable_x64", True)

NC = 2
NS = 16
L = 16
CH = 512


def _sc_scatter_build(N1, CPT, KB):
  RN = N1 // NS
  G = CPT // KB

  mesh = plsc.VectorSubcoreMesh(
      core_axis_name="c", subcore_axis_name="s", num_cores=NC,
      num_subcores=NS)

  @functools.partial(
      pl.kernel,
      out_type=jax.ShapeDtypeStruct((NC, N1), jnp.float32),
      mesh=mesh,
      compiler_params=pltpu.CompilerParams(
          needs_layout_passes=False, use_tc_tiling_on_sc=False),
      scratch_types=[
          pltpu.VMEM_SHARED((N1,), jnp.float32),
          pltpu.VMEM_SHARED((N1,), jnp.float32),
          pltpu.VMEM((RN,), jnp.float32),
          pltpu.VMEM((RN,), jnp.float32),
          pltpu.VMEM((RN,), jnp.float32),
          pltpu.VMEM((KB, CH), jnp.int32),
          pltpu.VMEM((KB, CH), jnp.int32),
          pltpu.VMEM((KB, CH), jnp.float32),
          pltpu.SemaphoreType.DMA,
          pltpu.SemaphoreType.DMA,
      ],
  )
  def sc_kernel(bel_hbm, pay_hbm, src_hbm, dst_hbm, t_out,
                table_sh, acc_sh, bel_v, pay_v, tbl_v,
                srcb, dstb, msgb, gsem, ssem):
    c = lax.axis_index("c")
    s = lax.axis_index("s")
    wid = c * NS + s
    base_n = s * RN

    pltpu.sync_copy(bel_hbm.at[pl.ds(base_n, RN)], bel_v)
    pltpu.sync_copy(pay_hbm.at[pl.ds(base_n, RN)], pay_v)

    zero16 = jnp.zeros((L,), jnp.float32)

    def build(_, off):
      b16 = bel_v[pl.ds(off, L)]
      p16 = pay_v[pl.ds(off, L)]
      m = jnp.maximum(jnp.sign(b16 - 0.5), 0.0)
      tbl_v[pl.ds(off, L)] = (2.0 * p16 - 10.0) * m
      return off + L

    lax.fori_loop(jnp.int32(0), jnp.int32(RN // L), build, jnp.int32(0))
    pltpu.sync_copy(tbl_v, table_sh.at[pl.ds(base_n, RN)])

    def zloop(_, off):
      tbl_v[pl.ds(off, L)] = zero16
      return off + L

    lax.fori_loop(jnp.int32(0), jnp.int32(RN // L), zloop, jnp.int32(0))
    pltpu.sync_copy(tbl_v, acc_sh.at[pl.ds(base_n, RN)])
    plsc.subcore_barrier()

    tile_row = wid * CPT

    def group(_, row0):
      row0a = pl.multiple_of(row0, 8)
      pltpu.sync_copy(src_hbm.at[pl.ds(row0a, KB)], srcb)
      pltpu.sync_copy(dst_hbm.at[pl.ds(row0a, KB)], dstb)
      ji = [jnp.int32(j) for j in range(KB)]
      gds = [pltpu.async_copy(table_sh.at[srcb.at[ji[j]]], msgb.at[ji[j]],
                              gsem)
             for j in range(KB)]
      sds = []
      for j in range(KB):
        gds[j].wait()
        sds.append(pltpu.async_copy(
            msgb.at[ji[j]], acc_sh.at[dstb.at[ji[j]]], ssem, add=True))
      for d in sds:
        d.wait()
      return row0 + KB

    lax.fori_loop(jnp.int32(0), jnp.int32(G), group, tile_row)
    plsc.subcore_barrier()

    pltpu.sync_copy(acc_sh.at[pl.ds(base_n, RN)],
                    t_out.at[c, pl.ds(base_n, RN)])

  return sc_kernel


def _tc_apply(b_ref, q_ref, t0_ref, t1_ref, o_ref):
  t = t0_ref[...] + t1_ref[...]
  b = b_ref[...]
  q = q_ref[...]
  d = t * (jnp.log(1.0 - q) - jnp.log(q))
  den = b + (1.0 - b) * jnp.exp(d)
  o_ref[...] = b / jnp.maximum(den, 1e-35)


def kernel(belief, probability, payoff_sample, edge_index):
  N = belief.shape[0]
  E = edge_index.shape[1]

  N1 = ((N + 1023) // 1024) * 1024
  CPT = 8 * math.ceil(E / (NC * NS * CH * 8))
  E1 = CPT * NC * NS * CH
  KB = max(k for k in range(8, 33, 8) if CPT % k == 0)

  f32 = jnp.float32
  bel_p = jnp.concatenate([belief.astype(f32), jnp.zeros((N1 - N,), f32)])
  q_p = jnp.concatenate(
      [probability.astype(f32), jnp.full((N1 - N,), 0.5, f32)])
  pay_p = jnp.concatenate(
      [payoff_sample.astype(f32), jnp.zeros((N1 - N,), f32)])
  pad_e = jnp.full((E1 - E,), N, jnp.int32)
  src2 = jnp.concatenate([edge_index[0].astype(jnp.int32), pad_e])
  dst2 = jnp.concatenate([edge_index[1].astype(jnp.int32), pad_e])
  src2 = src2.reshape(E1 // CH, CH)
  dst2 = dst2.reshape(E1 // CH, CH)

  sc = _sc_scatter_build(N1, CPT, KB)
  t_p = sc(bel_p, pay_p, src2, dst2)

  R = N1 // 128
  GB = max(g for g in range(1, 17)
           if R % g == 0 and (R // g) % 8 == 0)
  blk = (R // GB, 128)
  spec = pl.BlockSpec(blk, lambda i: (i, jnp.int32(0)))
  out = pl.pallas_call(
      _tc_apply,
      grid=(GB,),
      in_specs=[spec] * 4,
      out_specs=spec,
      out_shape=jax.ShapeDtypeStruct((R, 128), f32),
  )(bel_p.reshape(R, 128), q_p.reshape(R, 128),
    t_p[0].reshape(R, 128), t_p[1].reshape(R, 128))

  return out.reshape(N1)[:N].astype(jnp.float64)


# speedup vs baseline: 1.4239x; 1.1293x over previous
import functools
import math

import numpy as np

import jax
import jax.numpy as jnp
from jax import lax
from jax.experimental import pallas as pl
from jax.experimental.pallas import tpu as pltpu
from jax.experimental.pallas import tpu_sc as plsc

jax.config.update("jax_enable_x64", True)

NC = 2
NS = 16
L = 16
CH = 512


def _sc_scatter_build(N1, CPT, KB):
  RN = N1 // NS
  G = CPT // KB

  mesh = plsc.VectorSubcoreMesh(
      core_axis_name="c", subcore_axis_name="s", num_cores=NC,
      num_subcores=NS)

  @functools.partial(
      pl.kernel,
      out_type=jax.ShapeDtypeStruct((NC, N1), jnp.float32),
      mesh=mesh,
      compiler_params=pltpu.CompilerParams(
          needs_layout_passes=False, use_tc_tiling_on_sc=False),
      scratch_types=[
          pltpu.VMEM_SHARED((N1,), jnp.float32),
          pltpu.VMEM_SHARED((N1,), jnp.float32),
          pltpu.VMEM((RN,), jnp.float32),
          pltpu.VMEM((RN,), jnp.float32),
          pltpu.VMEM((RN,), jnp.float32),
          pltpu.VMEM((KB, CH), jnp.int32),
          pltpu.VMEM((KB, CH), jnp.int32),
          pltpu.VMEM((KB, CH), jnp.float32),
          pltpu.SemaphoreType.DMA,
          pltpu.SemaphoreType.DMA,
      ],
  )
  def sc_kernel(nodes_hbm, edges_hbm, t_out,
                table_sh, acc_sh, bel_v, pay_v, tbl_v,
                srcb, dstb, msgb, gsem, ssem):
    c = lax.axis_index("c")
    s = lax.axis_index("s")
    wid = c * NS + s
    base_n = s * RN
    i0, i1, i2 = jnp.int32(0), jnp.int32(1), jnp.int32(2)

    pltpu.sync_copy(nodes_hbm.at[i0, pl.ds(base_n, RN)], bel_v)
    pltpu.sync_copy(nodes_hbm.at[i2, pl.ds(base_n, RN)], pay_v)

    zero16 = jnp.zeros((L,), jnp.float32)

    def build(_, off):
      b16 = bel_v[pl.ds(off, L)]
      p16 = pay_v[pl.ds(off, L)]
      m = jnp.maximum(jnp.sign(b16 - 0.5), 0.0)
      tbl_v[pl.ds(off, L)] = (2.0 * p16 - 10.0) * m
      return off + L

    lax.fori_loop(jnp.int32(0), jnp.int32(RN // L), build, jnp.int32(0))
    pltpu.sync_copy(tbl_v, table_sh.at[pl.ds(base_n, RN)])

    def zloop(_, off):
      tbl_v[pl.ds(off, L)] = zero16
      return off + L

    lax.fori_loop(jnp.int32(0), jnp.int32(RN // L), zloop, jnp.int32(0))
    pltpu.sync_copy(tbl_v, acc_sh.at[pl.ds(base_n, RN)])
    plsc.subcore_barrier()

    tile_row = wid * CPT

    def group(_, row0):
      row0a = pl.multiple_of(row0, 8)
      pltpu.sync_copy(edges_hbm.at[i0, pl.ds(row0a, KB)], srcb)
      pltpu.sync_copy(edges_hbm.at[i1, pl.ds(row0a, KB)], dstb)
      ji = [jnp.int32(j) for j in range(KB)]
      gds = [pltpu.async_copy(table_sh.at[srcb.at[ji[j]]], msgb.at[ji[j]],
                              gsem)
             for j in range(KB)]
      sds = []
      for j in range(KB):
        gds[j].wait()
        sds.append(pltpu.async_copy(
            msgb.at[ji[j]], acc_sh.at[dstb.at[ji[j]]], ssem, add=True))
      for d in sds:
        d.wait()
      return row0 + KB

    lax.fori_loop(jnp.int32(0), jnp.int32(G), group, tile_row)
    plsc.subcore_barrier()

    pltpu.sync_copy(acc_sh.at[pl.ds(base_n, RN)],
                    t_out.at[c, pl.ds(base_n, RN)])

  return sc_kernel


def _tc_apply(b_ref, q_ref, t0_ref, t1_ref, o_ref):
  t = t0_ref[0] + t1_ref[0]
  b = b_ref[0]
  q = q_ref[0]
  d = t * (jnp.log(1.0 - q) - jnp.log(q))
  den = b + (1.0 - b) * jnp.exp(d)
  o_ref[...] = b / jnp.maximum(den, 1e-35)


def kernel(belief, probability, payoff_sample, edge_index):
  N = belief.shape[0]
  E = edge_index.shape[1]

  N1 = ((N + 1023) // 1024) * 1024
  CPT = 8 * math.ceil(E / (NC * NS * CH * 8))
  E1 = CPT * NC * NS * CH
  KB = max(k for k in range(8, 33, 8) if CPT % k == 0)

  f32 = jnp.float32
  nodes = jnp.stack([
      jnp.concatenate([belief.astype(f32), jnp.zeros((N1 - N,), f32)]),
      jnp.concatenate(
          [probability.astype(f32), jnp.full((N1 - N,), 0.5, f32)]),
      jnp.concatenate(
          [payoff_sample.astype(f32), jnp.zeros((N1 - N,), f32)]),
  ])
  edges = jnp.concatenate(
      [edge_index.astype(jnp.int32),
       jnp.full((2, E1 - E), N, jnp.int32)], axis=1)
  edges = edges.reshape(2, E1 // CH, CH)

  sc = _sc_scatter_build(N1, CPT, KB)
  t_p = sc(nodes, edges)

  R = N1 // 128
  GB = max(g for g in range(1, 17)
           if R % g == 0 and (R // g) % 8 == 0)
  blk = (1, R // GB, 128)
  z = np.int32(0)

  def _row_spec(r):
    return pl.BlockSpec(blk, lambda i, r=np.int32(r): (r, i, z))

  nodes3 = nodes.reshape(3, R, 128)
  t3 = t_p.reshape(2, R, 128)
  out = pl.pallas_call(
      _tc_apply,
      grid=(GB,),
      in_specs=[_row_spec(0), _row_spec(1), _row_spec(0), _row_spec(1)],
      out_specs=pl.BlockSpec((R // GB, 128), lambda i: (i, np.int32(0))),
      out_shape=jax.ShapeDtypeStruct((R, 128), f32),
  )(nodes3, nodes3, t3, t3)

  return out.reshape(N1)[:N].astype(jnp.float64)
